# Initial kernel scaffold; baseline (speedup 1.0000x reference)
import jax
import jax.numpy as jnp
from jax.experimental import pallas as pl


def kernel(x, weight, lora_A, lora_B):
    raise NotImplementedError("write your pallas kernel here")


# speedup vs baseline: 6.0734x; 6.0734x over previous
import functools

import jax
import jax.numpy as jnp
from jax import lax
from jax.experimental import pallas as pl
from jax.experimental.pallas import tpu as pltpu
from jax.experimental.pallas import tpu_sc as plsc

VOCAB = 1000000
EMBED_DIM = 32
RANK = 8
SCALING = 1.0 / 8.0
PACK = 4

MERGE_BLK = 2000


def _merge_body(w4_ref, a4_ref, bd_ref, out_ref):
    out_ref[...] = w4_ref[...] + jnp.dot(
        a4_ref[...], bd_ref[...], preferred_element_type=jnp.float32
    )


def _merge_table(weight4, lora_A4, bd):
    v4 = weight4.shape[0]
    d = PACK * EMBED_DIM
    return pl.pallas_call(
        _merge_body,
        grid=(v4 // MERGE_BLK,),
        in_specs=[
            pl.BlockSpec((MERGE_BLK, d), lambda i: (i, 0)),
            pl.BlockSpec((MERGE_BLK, PACK * RANK), lambda i: (i, 0)),
            pl.BlockSpec((PACK * RANK, d), lambda i: (0, 0)),
        ],
        out_specs=pl.BlockSpec((MERGE_BLK, d), lambda i: (i, 0)),
        out_shape=jax.ShapeDtypeStruct((v4, d), jnp.float32),
    )(weight4, lora_A4, bd)


def _gather_rows(table4, idx_flat, n_groups, group):
    info = plsc.get_sparse_core_info()
    nc, ns = info.num_cores, info.num_subcores
    nw = nc * ns
    gp_w = n_groups // nw
    gp_chunk = 16
    chunk = gp_chunk * group
    n_chunks = gp_w // gp_chunk
    mesh = plsc.VectorSubcoreMesh(core_axis_name="c", subcore_axis_name="s")

    @functools.partial(
        pl.kernel,
        mesh=mesh,
        out_type=jax.ShapeDtypeStruct((n_groups, group, EMBED_DIM), jnp.float32),
        scratch_types=[
            pltpu.VMEM((chunk,), jnp.int32),
            pltpu.VMEM((chunk,), jnp.int32),
            pltpu.VMEM((chunk, PACK * EMBED_DIM), jnp.float32),
            pltpu.VMEM((gp_chunk, group, EMBED_DIM), jnp.float32),
            pltpu.SemaphoreType.DMA,
        ],
    )
    def k(table_hbm, idx_hbm, out_hbm, idx_v, line_v, rows_v, out_v, sem):
        wid = lax.axis_index("s") * nc + lax.axis_index("c")
        g_base = wid * gp_w
        t_base = g_base * group

        def chunk_body(ci, carry):
            t_off = t_base + ci * chunk
            pltpu.sync_copy(idx_hbm.at[pl.ds(t_off, chunk)], idx_v)

            def split(i, c2):
                v = idx_v[pl.ds(i * 16, 16)]
                line_v[pl.ds(i * 16, 16)] = lax.shift_right_logical(v, 2)
                return c2

            lax.fori_loop(0, chunk // 16, split, 0, unroll=4)
            pltpu.async_copy(table_hbm.at[line_v], rows_v, sem).wait()

            def compact16(m, c2):
                base_t = m * 16
                qs = lax.shift_left(
                    jnp.bitwise_and(idx_v[pl.ds(base_t, 16)], PACK - 1), 5
                )
                for j in range(16):
                    t = base_t + j
                    g = t // group
                    c = lax.rem(t, group)
                    q = qs[j]
                    out_v[g, c, pl.ds(0, 16)] = rows_v[t, pl.ds(q, 16)]
                    out_v[g, c, pl.ds(16, 16)] = rows_v[t, pl.ds(q + 16, 16)]
                return c2

            lax.fori_loop(0, chunk // 16, compact16, 0)
            pltpu.sync_copy(
                out_v, out_hbm.at[pl.ds(g_base + ci * gp_chunk, gp_chunk)]
            )
            return carry

        lax.fori_loop(0, n_chunks, chunk_body, 0)

    return k(table4, idx_flat)


def kernel(x, weight, lora_A, lora_B):
    v4 = VOCAB // PACK
    weight4 = weight.reshape(v4, PACK * EMBED_DIM)
    lora_A4 = lora_A.reshape(v4, PACK * RANK)
    bd = jnp.kron(jnp.eye(PACK, dtype=jnp.float32), lora_B * SCALING)
    merged4 = _merge_table(weight4, lora_A4, bd)

    b, t, c = x.shape
    flat = x.reshape(-1).astype(jnp.int32)
    out = _gather_rows(merged4, flat, b * t, c)
    return out.reshape(b, t, c, EMBED_DIM)


# speedup vs baseline: 6.2367x; 1.0269x over previous
import functools

import jax
import jax.numpy as jnp
from jax import lax
from jax.experimental import pallas as pl
from jax.experimental.pallas import tpu as pltpu
from jax.experimental.pallas import tpu_sc as plsc

VOCAB = 1000000
EMBED_DIM = 32
RANK = 8
SCALING = 1.0 / 8.0
PACK = 4
V4 = VOCAB // PACK

MERGE_BLK = 2000


def _merge_body(w0, w1, w2, w3, a0, a1, a2, a3, bs_ref, out_ref):
    parts = []
    for wr, ar in ((w0, a0), (w1, a1), (w2, a2), (w3, a3)):
        parts.append(
            wr[...]
            + jnp.dot(ar[...], bs_ref[...], preferred_element_type=jnp.float32)
        )
    out_ref[...] = jnp.concatenate(parts, axis=1)


def _merge_table(weight, lora_A, bs):
    nblk = V4 // MERGE_BLK
    w_specs = [
        pl.BlockSpec((MERGE_BLK, EMBED_DIM), lambda i, k=k: (i + k * nblk, 0))
        for k in range(PACK)
    ]
    a_specs = [
        pl.BlockSpec((MERGE_BLK, RANK), lambda i, k=k: (i + k * nblk, 0))
        for k in range(PACK)
    ]
    return pl.pallas_call(
        _merge_body,
        grid=(nblk,),
        in_specs=w_specs + a_specs + [pl.BlockSpec((RANK, EMBED_DIM), lambda i: (0, 0))],
        out_specs=pl.BlockSpec((MERGE_BLK, PACK * EMBED_DIM), lambda i: (i, 0)),
        out_shape=jax.ShapeDtypeStruct((V4, PACK * EMBED_DIM), jnp.float32),
    )(weight, weight, weight, weight, lora_A, lora_A, lora_A, lora_A, bs)


def _gather_rows(table4, idx_flat, n_groups, group):
    info = plsc.get_sparse_core_info()
    nc, ns = info.num_cores, info.num_subcores
    nw = nc * ns
    gp_w = n_groups // nw
    gp_chunk = 16
    chunk = gp_chunk * group
    n_chunks = gp_w // gp_chunk
    mesh = plsc.VectorSubcoreMesh(core_axis_name="c", subcore_axis_name="s")

    @functools.partial(
        pl.kernel,
        mesh=mesh,
        out_type=jax.ShapeDtypeStruct((n_groups, group, EMBED_DIM), jnp.float32),
        scratch_types=[
            pltpu.VMEM((chunk,), jnp.int32),
            pltpu.VMEM((chunk,), jnp.int32),
            pltpu.VMEM((chunk,), jnp.int32),
            pltpu.VMEM((chunk, PACK * EMBED_DIM), jnp.float32),
            pltpu.VMEM((gp_chunk, group, EMBED_DIM), jnp.float32),
            pltpu.SemaphoreType.DMA,
        ],
    )
    def k(table_hbm, idx_hbm, out_hbm, idx_v, line_v, off_v, rows_v, out_v, sem):
        wid = lax.axis_index("s") * nc + lax.axis_index("c")
        g_base = wid * gp_w
        t_base = g_base * group

        def chunk_body(ci, carry):
            t_off = t_base + ci * chunk
            pltpu.sync_copy(idx_hbm.at[pl.ds(t_off, chunk)], idx_v)

            def split(i, c2):
                v = idx_v[pl.ds(i * 16, 16)]
                one = jnp.full((16,), 1, jnp.int32)
                zero = jnp.full((16,), 0, jnp.int32)
                q = (
                    jnp.where(v >= V4, one, zero)
                    + jnp.where(v >= 2 * V4, one, zero)
                    + jnp.where(v >= 3 * V4, one, zero)
                )
                line_v[pl.ds(i * 16, 16)] = v - q * V4
                off_v[pl.ds(i * 16, 16)] = lax.shift_left(q, 5)
                return c2

            lax.fori_loop(0, chunk // 16, split, 0, unroll=4)
            pltpu.async_copy(table_hbm.at[line_v], rows_v, sem).wait()

            def compact16(m, c2):
                base_t = m * 16
                qs = off_v[pl.ds(base_t, 16)]
                for j in range(16):
                    t = base_t + j
                    g = t // group
                    c = lax.rem(t, group)
                    q = qs[j]
                    out_v[g, c, pl.ds(0, 16)] = rows_v[t, pl.ds(q, 16)]
                    out_v[g, c, pl.ds(16, 16)] = rows_v[t, pl.ds(q + 16, 16)]
                return c2

            lax.fori_loop(0, chunk // 16, compact16, 0)
            pltpu.sync_copy(
                out_v, out_hbm.at[pl.ds(g_base + ci * gp_chunk, gp_chunk)]
            )
            return carry

        lax.fori_loop(0, n_chunks, chunk_body, 0)

    return k(table4, idx_flat)


def kernel(x, weight, lora_A, lora_B):
    merged4 = _merge_table(weight, lora_A, lora_B * SCALING)
    b, t, c = x.shape
    flat = x.reshape(-1).astype(jnp.int32)
    out = _gather_rows(merged4, flat, b * t, c)
    return out.reshape(b, t, c, EMBED_DIM)
